# baseline (speedup 1.0000x reference)
import functools

import jax
import jax.numpy as jnp
from jax import lax
from jax.experimental import pallas as pl
from jax.experimental.pallas import tpu as pltpu
from jax.experimental.pallas import tpu_sc as plsc

_N = 50000
_E = 800000
_Y = 64
_NPAD = 50176
_EPT = _E // 16
_W = 10000
_NCH = _EPT // _W
_SUBG_ROWS = _NPAD // 128


def _sc_body(dst0, ev0, dst1, ev1, part,
             acc, idxb0, valb0, idxb1, valb1, sem):
    cid = lax.axis_index("c")
    sid = lax.axis_index("s")

    def run_layer(dst, ev):
        base = sid * _EPT
        cps = (pltpu.async_copy(dst.at[pl.ds(base, _W)], idxb0, sem),
               pltpu.async_copy(ev.at[pl.ds(base, _W)], valb0, sem))

        def zero_body(k, carry):
            acc[pl.ds(k * 16, 16)] = jnp.zeros((16,), jnp.float32)
            return carry
        lax.fori_loop(0, _NPAD // 16, zero_body, 0, unroll=8)

        bufs = ((idxb0, valb0), (idxb1, valb1))
        for j in range(_NCH):
            cps[0].wait()
            cps[1].wait()
            ib, vb = bufs[j % 2]
            if j + 1 < _NCH:
                nib, nvb = bufs[(j + 1) % 2]
                off = base + (j + 1) * _W
                cps = (pltpu.async_copy(dst.at[pl.ds(off, _W)], nib, sem),
                       pltpu.async_copy(ev.at[pl.ds(off, _W)], nvb, sem))

            def scat_body(k, carry):
                iv = ib[pl.ds(k * 16, 16)]
                vv = vb[pl.ds(k * 16, 16)]
                plsc.addupdate_scatter(acc, [iv], vv)
                return carry
            lax.fori_loop(0, _W // 16, scat_body, 0, unroll=25)

        wid = cid * 16 + sid
        pltpu.sync_copy(acc, part.at[pl.ds(wid * _NPAD, _NPAD)])

    @pl.when(cid == 0)
    def _():
        run_layer(dst0, ev0)

    @pl.when(cid == 1)
    def _():
        run_layer(dst1, ev1)


@functools.cache
def _make_sc_call():
    return functools.partial(
        pl.kernel,
        mesh=plsc.VectorSubcoreMesh(core_axis_name="c", subcore_axis_name="s"),
        compiler_params=pltpu.CompilerParams(needs_layout_passes=False),
        out_type=[
            jax.ShapeDtypeStruct((32 * _NPAD,), jnp.float32),
        ],
        scratch_types=[
            pltpu.VMEM((_NPAD,), jnp.float32),
            pltpu.VMEM((_W,), jnp.int32),
            pltpu.VMEM((_W,), jnp.float32),
            pltpu.VMEM((_W,), jnp.int32),
            pltpu.VMEM((_W,), jnp.float32),
            pltpu.SemaphoreType.DMA,
        ],
    )(_sc_body)


def _tc_body(s_ref, w_ref, p_ref, h1_ref, h2_ref, cp_ref, aux_ref,
             subg_ref, uv_ref, out_ref, q_ref, scale_ref):
    li = pl.program_id(0)

    w = w_ref[...]
    t = jnp.maximum(w[:, 0:1] + w[:, 1:2], 0.0)
    c = t / jnp.maximum(jnp.sqrt(jnp.sum(t * t)), 1e-12)
    d = lax.dot_general(p_ref[...], c, (((0,), (0,)), ((), ())),
                        precision=lax.Precision.HIGHEST,
                        preferred_element_type=jnp.float32)
    nd = jnp.sqrt(jnp.sum(d * d))

    def mkscale(part):
        s_row = jnp.sum(part, axis=0, keepdims=True)
        return s_row / jnp.maximum(jnp.abs(s_row) * nd, 1e-12)

    scale_ref[pl.ds(li, 1), :] = mkscale(s_ref[0])
    out_ref[0] = d * scale_ref[pl.ds(li, 1), 0:_N]

    @pl.when(li == 1)
    def _():
        cw = _NPAD // 8
        ycol = lax.broadcasted_iota(jnp.int32, (_Y, 1), 0)

        def pbody(k, acc):
            iot = lax.broadcasted_iota(jnp.int32, (_Y, cw), 1) + k * cw
            cols = []
            for col in range(4):
                srow = scale_ref[pl.ds(col // 2, 1), pl.ds(k * cw, cw)]
                m = iot == uv_ref[:, col:col + 1]
                cols.append(jnp.sum(jnp.where(m, srow, 0.0), axis=1,
                                    keepdims=True))
            sv = subg_ref[0:1, pl.ds(k * cw, cw)]
            cols.append(jnp.sum(jnp.where(sv == ycol, 1.0, 0.0), axis=1,
                                keepdims=True))
            return acc + jnp.concatenate(cols, axis=1)
        picks = lax.fori_loop(0, 8, pbody, jnp.zeros((_Y, 5), jnp.float32))
        cnt = picks[:, 4:5]
        ys = cnt / jnp.maximum(cnt * nd, 1e-12)

        g = jnp.sum(d * cp_ref[...])
        d2 = d * d
        hv = lax.dot_general(h1_ref[...], d2, (((0,), (0,)), ((), ())),
                             precision=lax.Precision.HIGHEST,
                             preferred_element_type=jnp.float32)
        h2a = h2_ref[0:64, 0:1]
        pos = jnp.sum(jnp.maximum(hv, 0.0) * h2a)
        neg = jnp.sum(jnp.maximum(-hv, 0.0) * h2a)
        h2b = h2_ref[64:68, 0:1]
        h2bb = jnp.concatenate([h2b, h2b], axis=0)

        k0 = picks[:, 0:1] * picks[:, 1:2] * ys * g
        k1 = picks[:, 2:3] * picks[:, 3:4] * ys * g
        kq = (jnp.maximum(k0, 0.0) * pos + jnp.maximum(-k0, 0.0) * neg
              + jnp.maximum(k1, 0.0) * pos + jnp.maximum(-k1, 0.0) * neg)
        auxq = jnp.dot(aux_ref[...], h2bb,
                       precision=lax.Precision.HIGHEST,
                       preferred_element_type=jnp.float32)
        q_ref[...] = kq + auxq


_tc_call = pl.pallas_call(
    _tc_body,
    grid=(2, 1),
    in_specs=[
        pl.BlockSpec((1, 16, _NPAD), lambda l, i: (l, 0, 0)),
        pl.BlockSpec((64, 2), lambda l, i: (0, 0)),
        pl.BlockSpec((64, 64), lambda l, i: (0, 0)),
        pl.BlockSpec((64, 64), lambda l, i: (0, 0)),
        pl.BlockSpec((68, 1), lambda l, i: (0, 0)),
        pl.BlockSpec((64, 1), lambda l, i: (0, 0)),
        pl.BlockSpec((_Y, 8), lambda l, i: (0, 0)),
        pl.BlockSpec((1, _NPAD), lambda l, i: (0, 0)),
        pl.BlockSpec((_Y, 4), lambda l, i: (0, 0)),
    ],
    out_specs=[
        pl.BlockSpec((1, 64, _N), lambda l, i: (l, 0, 0)),
        pl.BlockSpec((_Y, 1), lambda l, i: (0, 0)),
    ],
    out_shape=[
        jax.ShapeDtypeStruct((2, 64, _N), jnp.float32),
        jax.ShapeDtypeStruct((_Y, 1), jnp.float32),
    ],
    scratch_shapes=[pltpu.VMEM((2, _NPAD), jnp.float32)],
)


def kernel(edge_index0, edge_value0, edge_index1, edge_value1, subg_rows,
           action_cols, aux_input, w_n2l, p_node_conv, h1_weight, h2_weight,
           cross_product):
    ei0f = edge_index0.reshape(-1)
    ei1f = edge_index1.reshape(-1)
    (part,) = jax.tree.leaves(
        _make_sc_call()(ei0f, edge_value0, ei1f, edge_value1))
    s = part.reshape(2, 16, _NPAD)
    uv = jnp.stack([ei0f[action_cols], ei0f[action_cols + _E],
                    ei1f[action_cols], ei1f[action_cols + _E]],
                   axis=1)
    subg_p = jnp.concatenate(
        [subg_rows, jnp.full((_NPAD - _N,), _Y, subg_rows.dtype)]
    ).reshape(1, _NPAD)
    aux8 = aux_input.reshape(_Y, 8)
    cur_ml_t, q = _tc_call(s, w_n2l.T, p_node_conv, h1_weight, h2_weight,
                           cross_product, aux8, subg_p, uv)
    return (q, jnp.swapaxes(cur_ml_t, 1, 2))


# speedup vs baseline: 1.1069x; 1.1069x over previous
import functools

import jax
import jax.numpy as jnp
from jax import lax
from jax.experimental import pallas as pl
from jax.experimental.pallas import tpu as pltpu
from jax.experimental.pallas import tpu_sc as plsc

_N = 50000
_E = 800000
_Y = 64
_NPAD = 50176
_EPT = _E // 16
_W = 10000
_NCH = _EPT // _W
_SUBG_ROWS = _NPAD // 128


def _sc_body(dst0, ev0, dst1, ev1, part,
             acc, idxb0, valb0, idxb1, valb1, sem):
    cid = lax.axis_index("c")
    sid = lax.axis_index("s")

    def run_layer(dst, ev):
        base = sid * _EPT
        cps = (pltpu.async_copy(dst.at[pl.ds(base, _W)], idxb0, sem),
               pltpu.async_copy(ev.at[pl.ds(base, _W)], valb0, sem))

        def zero_body(k, carry):
            acc[pl.ds(k * 16, 16)] = jnp.zeros((16,), jnp.float32)
            return carry
        lax.fori_loop(0, _NPAD // 16, zero_body, 0, unroll=8)

        bufs = ((idxb0, valb0), (idxb1, valb1))
        for j in range(_NCH):
            cps[0].wait()
            cps[1].wait()
            ib, vb = bufs[j % 2]
            if j + 1 < _NCH:
                nib, nvb = bufs[(j + 1) % 2]
                off = base + (j + 1) * _W
                cps = (pltpu.async_copy(dst.at[pl.ds(off, _W)], nib, sem),
                       pltpu.async_copy(ev.at[pl.ds(off, _W)], nvb, sem))

            def scat_body(k, carry):
                iv = ib[pl.ds(k * 16, 16)]
                vv = vb[pl.ds(k * 16, 16)]
                plsc.addupdate_scatter(acc, [iv], vv)
                return carry
            lax.fori_loop(0, _W // 16, scat_body, 0, unroll=5)

        wid = cid * 16 + sid
        pltpu.sync_copy(acc, part.at[pl.ds(wid * _NPAD, _NPAD)])

    @pl.when(cid == 0)
    def _():
        run_layer(dst0, ev0)

    @pl.when(cid == 1)
    def _():
        run_layer(dst1, ev1)


@functools.cache
def _make_sc_call():
    return functools.partial(
        pl.kernel,
        mesh=plsc.VectorSubcoreMesh(core_axis_name="c", subcore_axis_name="s"),
        compiler_params=pltpu.CompilerParams(needs_layout_passes=False),
        out_type=[
            jax.ShapeDtypeStruct((32 * _NPAD,), jnp.float32),
        ],
        scratch_types=[
            pltpu.VMEM((_NPAD,), jnp.float32),
            pltpu.VMEM((_W,), jnp.int32),
            pltpu.VMEM((_W,), jnp.float32),
            pltpu.VMEM((_W,), jnp.int32),
            pltpu.VMEM((_W,), jnp.float32),
            pltpu.SemaphoreType.DMA,
        ],
    )(_sc_body)


def _tc_body(s_ref, w_ref, p_ref, h1_ref, h2_ref, cp_ref, aux_ref,
             subg_ref, uv_ref, out_ref, q_ref, scale_ref):
    li = pl.program_id(0)

    w = w_ref[...]
    t = jnp.maximum(w[:, 0:1] + w[:, 1:2], 0.0)
    c = t / jnp.maximum(jnp.sqrt(jnp.sum(t * t)), 1e-12)
    d = lax.dot_general(p_ref[...], c, (((0,), (0,)), ((), ())),
                        precision=lax.Precision.HIGHEST,
                        preferred_element_type=jnp.float32)
    nd = jnp.sqrt(jnp.sum(d * d))

    p3 = jnp.reshape(s_ref[...], (16, _NPAD // 128, 128))
    s_row = jnp.reshape(jnp.sum(p3, axis=0), (1, _NPAD))
    scale_ref[pl.ds(li, 1), :] = (
        s_row / jnp.maximum(jnp.abs(s_row) * nd, 1e-12))
    out_ref[0] = d * scale_ref[pl.ds(li, 1), 0:_N]

    @pl.when(li == 1)
    def _():
        cw = _NPAD // 8
        ycol = lax.broadcasted_iota(jnp.int32, (_Y, 1), 0)

        def pbody(k, acc):
            iot = lax.broadcasted_iota(jnp.int32, (_Y, cw), 1) + k * cw
            cols = []
            for col in range(4):
                srow = scale_ref[pl.ds(col // 2, 1), pl.ds(k * cw, cw)]
                m = iot == uv_ref[:, col:col + 1]
                cols.append(jnp.sum(jnp.where(m, srow, 0.0), axis=1,
                                    keepdims=True))
            sv = subg_ref[0:1, pl.ds(k * cw, cw)]
            cols.append(jnp.sum(jnp.where(sv == ycol, 1.0, 0.0), axis=1,
                                keepdims=True))
            return acc + jnp.concatenate(cols, axis=1)
        picks = lax.fori_loop(0, 8, pbody, jnp.zeros((_Y, 5), jnp.float32))
        cnt = picks[:, 4:5]
        ys = cnt / jnp.maximum(cnt * nd, 1e-12)

        g = jnp.sum(d * cp_ref[...])
        d2 = d * d
        hv = lax.dot_general(h1_ref[...], d2, (((0,), (0,)), ((), ())),
                             precision=lax.Precision.HIGHEST,
                             preferred_element_type=jnp.float32)
        h2a = h2_ref[0:64, 0:1]
        pos = jnp.sum(jnp.maximum(hv, 0.0) * h2a)
        neg = jnp.sum(jnp.maximum(-hv, 0.0) * h2a)
        h2b = h2_ref[64:68, 0:1]
        h2bb = jnp.concatenate([h2b, h2b], axis=0)

        k0 = picks[:, 0:1] * picks[:, 1:2] * ys * g
        k1 = picks[:, 2:3] * picks[:, 3:4] * ys * g
        kq = (jnp.maximum(k0, 0.0) * pos + jnp.maximum(-k0, 0.0) * neg
              + jnp.maximum(k1, 0.0) * pos + jnp.maximum(-k1, 0.0) * neg)
        auxq = jnp.dot(aux_ref[...], h2bb,
                       precision=lax.Precision.HIGHEST,
                       preferred_element_type=jnp.float32)
        q_ref[...] = kq + auxq


_tc_call = pl.pallas_call(
    _tc_body,
    grid=(2, 1),
    in_specs=[
        pl.BlockSpec((16 * _NPAD // 128, 128), lambda l, i: (l, 0)),
        pl.BlockSpec((64, 2), lambda l, i: (0, 0)),
        pl.BlockSpec((64, 64), lambda l, i: (0, 0)),
        pl.BlockSpec((64, 64), lambda l, i: (0, 0)),
        pl.BlockSpec((68, 1), lambda l, i: (0, 0)),
        pl.BlockSpec((64, 1), lambda l, i: (0, 0)),
        pl.BlockSpec((_Y, 8), lambda l, i: (0, 0)),
        pl.BlockSpec((1, _NPAD), lambda l, i: (0, 0)),
        pl.BlockSpec((_Y, 4), lambda l, i: (0, 0)),
    ],
    out_specs=[
        pl.BlockSpec((1, 64, _N), lambda l, i: (l, 0, 0)),
        pl.BlockSpec((_Y, 1), lambda l, i: (0, 0)),
    ],
    out_shape=[
        jax.ShapeDtypeStruct((2, 64, _N), jnp.float32),
        jax.ShapeDtypeStruct((_Y, 1), jnp.float32),
    ],
    scratch_shapes=[pltpu.VMEM((2, _NPAD), jnp.float32)],
)


def kernel(edge_index0, edge_value0, edge_index1, edge_value1, subg_rows,
           action_cols, aux_input, w_n2l, p_node_conv, h1_weight, h2_weight,
           cross_product):
    ei0f = edge_index0.reshape(-1)
    ei1f = edge_index1.reshape(-1)
    (part,) = jax.tree.leaves(
        _make_sc_call()(ei0f, edge_value0, ei1f, edge_value1))
    s = part.reshape(32 * _NPAD // 128, 128)
    uv = jnp.stack([ei0f[action_cols], ei0f[action_cols + _E],
                    ei1f[action_cols], ei1f[action_cols + _E]],
                   axis=1)
    subg_p = jnp.concatenate(
        [subg_rows, jnp.full((_NPAD - _N,), _Y, subg_rows.dtype)]
    ).reshape(1, _NPAD)
    aux8 = aux_input.reshape(_Y, 8)
    cur_ml_t, q = _tc_call(s, w_n2l.T, p_node_conv, h1_weight, h2_weight,
                           cross_product, aux8, subg_p, uv)
    return (q, jnp.swapaxes(cur_ml_t, 1, 2))
